# baseline (speedup 1.0000x reference)
import jax
import jax.numpy as jnp
from jax.experimental import pallas as pl
from jax.experimental.pallas import tpu as pltpu

F32 = jnp.float32
BF16 = jnp.bfloat16
I32 = jnp.int32
LANE = 128


def _dot(a, b):
    return jax.lax.dot_general(a, b, (((a.ndim - 1,), (0,)), ((), ())),
                               preferred_element_type=F32)


def _gelu(v):
    return 0.5 * v * (1.0 + jax.lax.erf(v * 0.7071067811865476))


def _pre_kernel(x_ref, wt_ref, bt_ref, xe_ref):
    xe_ref[...] = _gelu(_dot(x_ref[...], wt_ref[...]) + bt_ref[...])


def _moe_kernel(xe_ref, emb_ref, wg_ref, bgp_ref, w1c_ref, b1c_ref, w2c_ref,
                bband_ref, b2p_ref, out_ref, w1s, w2s, sem1, sem2):
    i = pl.program_id(0)

    @pl.when(i == 0)
    def _load_weights():
        pltpu.make_async_copy(w1c_ref, w1s, sem1).start()
        pltpu.make_async_copy(w2c_ref, w2s, sem2).start()
        pltpu.make_async_copy(w1c_ref, w1s, sem1).wait()
        pltpu.make_async_copy(w2c_ref, w2s, sem2).wait()

    bm = xe_ref.shape[0]
    lanes = jax.lax.broadcasted_iota(I32, (bm, LANE), 1)
    xf = xe_ref[...] + emb_ref[0]
    xbv = xf.astype(BF16)
    logits = _dot(xbv, wg_ref[...]) + bgp_ref[...]
    m = jnp.max(logits, axis=1, keepdims=True)
    ex = jnp.exp(logits - m)
    probs = ex / jnp.sum(ex, axis=1, keepdims=True)
    v1 = jnp.max(probs, axis=1, keepdims=True)
    f1 = jnp.min(jnp.where(probs == v1, lanes, LANE), axis=1, keepdims=True)
    sel1 = lanes == f1
    p2 = jnp.where(sel1, -1.0, probs)
    v2 = jnp.max(p2, axis=1, keepdims=True)
    f2 = jnp.min(jnp.where(p2 == v2, lanes, LANE), axis=1, keepdims=True)
    sel2 = lanes == f2
    wd = jnp.where(sel1, v1, 0.0) + jnp.where(sel2, v2, 0.0)
    wdb = wd.astype(BF16)

    h = _gelu(_dot(xbv, w1s[...]) + b1c_ref[...])
    wband = _dot(wdb, bband_ref[...])
    wh = (h * wband).astype(BF16)
    out_ref[...] = _dot(wh, w2s[...]) + _dot(wdb, b2p_ref[...])


def _post_kernel(moe_ref, xe_ref, emb_ref, g_ref, be_ref, wp_ref, bp_ref,
                 out_ref):
    mo = moe_ref[...]
    m = jnp.mean(mo, axis=1, keepdims=True)
    v = jnp.mean((mo - m) ** 2, axis=1, keepdims=True)
    ln = (mo - m) / jnp.sqrt(v + 1e-5) * g_ref[0] + be_ref[0]
    y = _gelu(ln) + (xe_ref[...] + emb_ref[0])
    out_ref[...] = _dot(y.astype(BF16), wp_ref[0]) + bp_ref[0]


def kernel(x, W_txt, b_txt, l2_emb, cl_emb, Wg, bg, W1, b1, W2, b2,
           g_l2, be_l2, g_cl, be_cl, W_t2v, b_t2v, W_cl, b_cl):
    B, S, TD = x.shape
    H = W_txt.shape[1]
    E = Wg.shape[1]
    SD = W_t2v.shape[1]
    EH = E * H
    T = B * S
    TT = 2 * T
    bma = min(1024, T)
    nra = T // bma
    bm = min(256, T)
    nrb = T // bm
    bmc = min(1024, T)
    nrc = T // bmc

    xf = x.reshape(T, TD).astype(BF16)

    xe = pl.pallas_call(
        _pre_kernel,
        grid=(nra,),
        in_specs=[
            pl.BlockSpec((bma, TD), lambda i: (i, 0)),
            pl.BlockSpec((TD, H), lambda i: (0, 0)),
            pl.BlockSpec((1, H), lambda i: (0, 0)),
        ],
        out_specs=pl.BlockSpec((bma, H), lambda i: (i, 0)),
        out_shape=jax.ShapeDtypeStruct((T, H), F32),
    )(xf, W_txt.astype(BF16), b_txt.reshape(1, H))

    emb = jnp.concatenate([l2_emb.reshape(1, 1, H), cl_emb.reshape(1, 1, H)],
                          axis=0)
    wg_pad = jnp.zeros((H, LANE), F32).at[:, :E].set(Wg).astype(BF16)
    bg_pad = jnp.full((1, LANE), -1e30, F32).at[0, :E].set(bg)
    w1c = jnp.transpose(W1, (1, 0, 2)).reshape(H, EH).astype(BF16)
    b1c = b1.reshape(1, EH)
    w2c = W2.reshape(EH, H).astype(BF16)
    bband = jnp.zeros((LANE, EH), F32).at[:E].set(
        jnp.repeat(jnp.eye(E, dtype=F32), H, axis=1)).astype(BF16)
    b2p = jnp.zeros((LANE, H), F32).at[:E].set(b2).astype(BF16)

    moe = pl.pallas_call(
        _moe_kernel,
        grid=(2 * nrb,),
        in_specs=[
            pl.BlockSpec((bm, H), lambda i: (i % nrb, 0)),
            pl.BlockSpec((1, 1, H), lambda i: (i // nrb, 0, 0)),
            pl.BlockSpec((H, LANE), lambda i: (0, 0)),
            pl.BlockSpec((1, LANE), lambda i: (0, 0)),
            pl.BlockSpec(memory_space=pl.ANY),
            pl.BlockSpec((1, EH), lambda i: (0, 0)),
            pl.BlockSpec(memory_space=pl.ANY),
            pl.BlockSpec((LANE, EH), lambda i: (0, 0)),
            pl.BlockSpec((LANE, H), lambda i: (0, 0)),
        ],
        out_specs=pl.BlockSpec((bm, H), lambda i: (i, 0)),
        out_shape=jax.ShapeDtypeStruct((TT, H), F32),
        scratch_shapes=[
            pltpu.VMEM((H, EH), BF16),
            pltpu.VMEM((EH, H), BF16),
            pltpu.SemaphoreType.DMA,
            pltpu.SemaphoreType.DMA,
        ],
        compiler_params=pltpu.CompilerParams(
            dimension_semantics=("arbitrary",)),
    )(xe, emb, wg_pad, bg_pad, w1c, b1c, w2c, bband, b2p)

    g2 = jnp.concatenate([g_l2.reshape(1, 1, H), g_cl.reshape(1, 1, H)], 0)
    be2 = jnp.concatenate([be_l2.reshape(1, 1, H), be_cl.reshape(1, 1, H)], 0)
    wp = jnp.stack([W_t2v, W_cl], axis=0).astype(BF16)
    bp = jnp.concatenate([b_t2v.reshape(1, 1, SD), b_cl.reshape(1, 1, H)], 0)

    out = pl.pallas_call(
        _post_kernel,
        grid=(2 * nrc,),
        in_specs=[
            pl.BlockSpec((bmc, H), lambda i: (i, 0)),
            pl.BlockSpec((bmc, H), lambda i: (i % nrc, 0)),
            pl.BlockSpec((1, 1, H), lambda i: (i // nrc, 0, 0)),
            pl.BlockSpec((1, 1, H), lambda i: (i // nrc, 0, 0)),
            pl.BlockSpec((1, 1, H), lambda i: (i // nrc, 0, 0)),
            pl.BlockSpec((1, H, H), lambda i: (i // nrc, 0, 0)),
            pl.BlockSpec((1, 1, H), lambda i: (i // nrc, 0, 0)),
        ],
        out_specs=pl.BlockSpec((bmc, H), lambda i: (i, 0)),
        out_shape=jax.ShapeDtypeStruct((TT, H), F32),
    )(moe, xe, emb, g2, be2, wp, bp)

    return (out[:T].reshape(B, S, SD), out[T:].reshape(B, S, H))


# speedup vs baseline: 1.0128x; 1.0128x over previous
import jax
import jax.numpy as jnp
from jax.experimental import pallas as pl
from jax.experimental.pallas import tpu as pltpu

F32 = jnp.float32
BF16 = jnp.bfloat16
I32 = jnp.int32
LANE = 128


def _dot(a, b):
    return jax.lax.dot_general(a, b, (((a.ndim - 1,), (0,)), ((), ())),
                               preferred_element_type=F32)


def _gelu(v):
    return 0.5 * v * (1.0 + jax.lax.erf(v * 0.7071067811865476))


def _pre_kernel(x_ref, wt_ref, bt_ref, xe_ref):
    xe_ref[...] = _gelu(_dot(x_ref[...], wt_ref[...]) + bt_ref[...])


def _moe_kernel(xe_ref, emb_ref, wg_ref, bgp_ref, w1c_ref, b1c_ref, w2c_ref,
                bband_ref, b2p_ref, out_ref, w1s, w2s, sem1, sem2):
    i = pl.program_id(0)

    @pl.when(i == 0)
    def _load_weights():
        pltpu.make_async_copy(w1c_ref, w1s, sem1).start()
        pltpu.make_async_copy(w2c_ref, w2s, sem2).start()
        pltpu.make_async_copy(w1c_ref, w1s, sem1).wait()
        pltpu.make_async_copy(w2c_ref, w2s, sem2).wait()

    bm = xe_ref.shape[0]
    lanes = jax.lax.broadcasted_iota(I32, (bm, LANE), 1)
    xf = xe_ref[...] + emb_ref[0]
    xbv = xf.astype(BF16)
    logits = _dot(xbv, wg_ref[...]) + bgp_ref[...]
    m = jnp.max(logits, axis=1, keepdims=True)
    ex = jnp.exp(logits - m)
    probs = ex / jnp.sum(ex, axis=1, keepdims=True)
    v1 = jnp.max(probs, axis=1, keepdims=True)
    f1 = jnp.min(jnp.where(probs == v1, lanes, LANE), axis=1, keepdims=True)
    sel1 = lanes == f1
    p2 = jnp.where(sel1, -1.0, probs)
    v2 = jnp.max(p2, axis=1, keepdims=True)
    f2 = jnp.min(jnp.where(p2 == v2, lanes, LANE), axis=1, keepdims=True)
    sel2 = lanes == f2
    wd = jnp.where(sel1, v1, 0.0) + jnp.where(sel2, v2, 0.0)
    wdb = wd.astype(BF16)

    eh = w1s.shape[1]
    half = eh // 2
    acc = _dot(wdb, b2p_ref[...])
    for k in range(2):
        sl = slice(k * half, (k + 1) * half)
        hk = _gelu(_dot(xbv, w1s[:, sl]) + b1c_ref[:, sl])
        wbk = _dot(wdb, bband_ref[:, sl])
        acc = acc + _dot((hk * wbk).astype(BF16), w2s[sl, :])
    out_ref[...] = acc


def _post_kernel(moe_ref, xe_ref, emb_ref, g_ref, be_ref, wp_ref, bp_ref,
                 out_ref):
    mo = moe_ref[...]
    m = jnp.mean(mo, axis=1, keepdims=True)
    v = jnp.mean((mo - m) ** 2, axis=1, keepdims=True)
    ln = (mo - m) / jnp.sqrt(v + 1e-5) * g_ref[0] + be_ref[0]
    y = _gelu(ln) + (xe_ref[...] + emb_ref[0])
    out_ref[...] = _dot(y.astype(BF16), wp_ref[0]) + bp_ref[0]


def kernel(x, W_txt, b_txt, l2_emb, cl_emb, Wg, bg, W1, b1, W2, b2,
           g_l2, be_l2, g_cl, be_cl, W_t2v, b_t2v, W_cl, b_cl):
    B, S, TD = x.shape
    H = W_txt.shape[1]
    E = Wg.shape[1]
    SD = W_t2v.shape[1]
    EH = E * H
    T = B * S
    TT = 2 * T
    bma = min(1024, T)
    nra = T // bma
    bm = min(512, T)
    nrb = T // bm
    bmc = min(1024, T)
    nrc = T // bmc

    xf = x.reshape(T, TD).astype(BF16)

    xe = pl.pallas_call(
        _pre_kernel,
        grid=(nra,),
        in_specs=[
            pl.BlockSpec((bma, TD), lambda i: (i, 0)),
            pl.BlockSpec((TD, H), lambda i: (0, 0)),
            pl.BlockSpec((1, H), lambda i: (0, 0)),
        ],
        out_specs=pl.BlockSpec((bma, H), lambda i: (i, 0)),
        out_shape=jax.ShapeDtypeStruct((T, H), F32),
    )(xf, W_txt.astype(BF16), b_txt.reshape(1, H))

    emb = jnp.concatenate([l2_emb.reshape(1, 1, H), cl_emb.reshape(1, 1, H)],
                          axis=0)
    wg_pad = jnp.zeros((H, LANE), F32).at[:, :E].set(Wg).astype(BF16)
    bg_pad = jnp.full((1, LANE), -1e30, F32).at[0, :E].set(bg)
    w1c = jnp.transpose(W1, (1, 0, 2)).reshape(H, EH).astype(BF16)
    b1c = b1.reshape(1, EH)
    w2c = W2.reshape(EH, H).astype(BF16)
    bband = jnp.zeros((LANE, EH), F32).at[:E].set(
        jnp.repeat(jnp.eye(E, dtype=F32), H, axis=1)).astype(BF16)
    b2p = jnp.zeros((LANE, H), F32).at[:E].set(b2).astype(BF16)

    moe = pl.pallas_call(
        _moe_kernel,
        grid=(2 * nrb,),
        in_specs=[
            pl.BlockSpec((bm, H), lambda i: (i % nrb, 0)),
            pl.BlockSpec((1, 1, H), lambda i: (i // nrb, 0, 0)),
            pl.BlockSpec((H, LANE), lambda i: (0, 0)),
            pl.BlockSpec((1, LANE), lambda i: (0, 0)),
            pl.BlockSpec(memory_space=pl.ANY),
            pl.BlockSpec((1, EH), lambda i: (0, 0)),
            pl.BlockSpec(memory_space=pl.ANY),
            pl.BlockSpec((LANE, EH), lambda i: (0, 0)),
            pl.BlockSpec((LANE, H), lambda i: (0, 0)),
        ],
        out_specs=pl.BlockSpec((bm, H), lambda i: (i, 0)),
        out_shape=jax.ShapeDtypeStruct((TT, H), F32),
        scratch_shapes=[
            pltpu.VMEM((H, EH), BF16),
            pltpu.VMEM((EH, H), BF16),
            pltpu.SemaphoreType.DMA,
            pltpu.SemaphoreType.DMA,
        ],
        compiler_params=pltpu.CompilerParams(
            dimension_semantics=("arbitrary",)),
    )(xe, emb, wg_pad, bg_pad, w1c, b1c, w2c, bband, b2p)

    g2 = jnp.concatenate([g_l2.reshape(1, 1, H), g_cl.reshape(1, 1, H)], 0)
    be2 = jnp.concatenate([be_l2.reshape(1, 1, H), be_cl.reshape(1, 1, H)], 0)
    wp = jnp.stack([W_t2v, W_cl], axis=0).astype(BF16)
    bp = jnp.concatenate([b_t2v.reshape(1, 1, SD), b_cl.reshape(1, 1, H)], 0)

    out = pl.pallas_call(
        _post_kernel,
        grid=(2 * nrc,),
        in_specs=[
            pl.BlockSpec((bmc, H), lambda i: (i, 0)),
            pl.BlockSpec((bmc, H), lambda i: (i % nrc, 0)),
            pl.BlockSpec((1, 1, H), lambda i: (i // nrc, 0, 0)),
            pl.BlockSpec((1, 1, H), lambda i: (i // nrc, 0, 0)),
            pl.BlockSpec((1, 1, H), lambda i: (i // nrc, 0, 0)),
            pl.BlockSpec((1, H, H), lambda i: (i // nrc, 0, 0)),
            pl.BlockSpec((1, 1, H), lambda i: (i // nrc, 0, 0)),
        ],
        out_specs=pl.BlockSpec((bmc, H), lambda i: (i, 0)),
        out_shape=jax.ShapeDtypeStruct((TT, H), F32),
    )(moe, xe, emb, g2, be2, wp, bp)

    return (out[:T].reshape(B, S, SD), out[T:].reshape(B, S, H))
